# baseline (speedup 1.0000x reference)
import functools
import math

import jax
import jax.numpy as jnp
from jax import lax
from jax.experimental import pallas as pl
from jax.experimental.pallas import tpu as pltpu
from jax.experimental.pallas import tpu_sc as plsc

N = 10000
E = 160000
D = 256
H = 8
DH = 32
ED = 16
FF = 1024

NC = 2
NS = 16
LN_EPS = 1e-6
INV_SQRT_DH = 1.0 / math.sqrt(DH)

W = 25
PW = 32
EPS_ = E // NS
NWIN = EPS_ // W
AW = 144
QW = 192
EW = 144


def _tc1_body(x_ref, c_ref, Wa_ref, ba_ref, Wq_ref, bq_ref, Wkvh_ref, bkv_ref,
              Wq2e_ref, q_ref, qe_ref, nkv_ref, modt_ref):
    cc = c_ref[...]
    sc = cc * jax.nn.sigmoid(cc)
    mod = jnp.dot(sc, Wa_ref[...], preferred_element_type=jnp.float32) + ba_ref[...]
    xx = x_ref[...]
    mu = jnp.mean(xx, axis=-1, keepdims=True)
    var = jnp.mean((xx - mu) * (xx - mu), axis=-1, keepdims=True)
    ln = (xx - mu) * lax.rsqrt(var + LN_EPS)
    h = ln * (1.0 + mod[:, D:2 * D]) + mod[:, 0:D]
    q = jnp.dot(h, Wq_ref[...], preferred_element_type=jnp.float32) + bq_ref[...]
    q_ref[...] = q
    qe_ref[...] = jnp.dot(q, Wq2e_ref[...], preferred_element_type=jnp.float32)
    nkv_ref[...] = jnp.dot(h, Wkvh_ref[...], preferred_element_type=jnp.float32) + bkv_ref[...]
    modt_ref[...] = mod[:, 2 * D:6 * D]


def _tc1(x, c, Wa, ba, Wq, bq, Wkvh, bkv, Wq2e):
    BN = 1000
    grid = (N // BN,)
    return pl.pallas_call(
        _tc1_body,
        grid=grid,
        in_specs=[
            pl.BlockSpec((BN, D), lambda i: (i, 0)),
            pl.BlockSpec((BN, D), lambda i: (i, 0)),
            pl.BlockSpec((D, 6 * D), lambda i: (0, 0)),
            pl.BlockSpec((1, 6 * D), lambda i: (0, 0)),
            pl.BlockSpec((D, D), lambda i: (0, 0)),
            pl.BlockSpec((1, D), lambda i: (0, 0)),
            pl.BlockSpec((D, 2 * D), lambda i: (0, 0)),
            pl.BlockSpec((1, 2 * D), lambda i: (0, 0)),
            pl.BlockSpec((D, H * ED), lambda i: (0, 0)),
        ],
        out_specs=[
            pl.BlockSpec((BN, D), lambda i: (i, 0)),
            pl.BlockSpec((BN, H * ED), lambda i: (i, 0)),
            pl.BlockSpec((BN, 2 * D), lambda i: (i, 0)),
            pl.BlockSpec((BN, 4 * D), lambda i: (i, 0)),
        ],
        out_shape=[
            jax.ShapeDtypeStruct((N, D), jnp.float32),
            jax.ShapeDtypeStruct((N, H * ED), jnp.float32),
            jax.ShapeDtypeStruct((N, 2 * D), jnp.float32),
            jax.ShapeDtypeStruct((N, 4 * D), jnp.float32),
        ],
    )(x, c, Wa, ba, Wq, bq, Wkvh, bkv, Wq2e)


def _tcev_body(ea_ref, Wv_ref, e0_ref, e1_ref):
    ea = ea_ref[...]
    ev = jnp.dot(ea, Wv_ref[...], preferred_element_type=jnp.float32)
    e0_ref[...] = jnp.concatenate([ea, ev[:, :128]], axis=-1)
    e1_ref[...] = jnp.concatenate([ea, ev[:, 128:]], axis=-1)


def _tcev(edge_attr, Wvecat):
    BE = 4000
    return pl.pallas_call(
        _tcev_body,
        grid=(E // BE,),
        in_specs=[
            pl.BlockSpec((BE, ED), lambda i: (i, 0)),
            pl.BlockSpec((ED, D), lambda i: (0, 0)),
        ],
        out_specs=[pl.BlockSpec((BE, EW), lambda i: (i, 0)),
                   pl.BlockSpec((BE, EW), lambda i: (i, 0))],
        out_shape=[jax.ShapeDtypeStruct((E, EW), jnp.float32),
                   jax.ShapeDtypeStruct((E, EW), jnp.float32)],
    )(edge_attr, Wvecat)


_sc_mesh = plsc.VectorSubcoreMesh(
    core_axis_name="c", subcore_axis_name="s", num_cores=NC, num_subcores=NS)


@functools.partial(
    pl.kernel,
    out_type=(jax.ShapeDtypeStruct((N, AW), jnp.float32),
              jax.ShapeDtypeStruct((N, AW), jnp.float32)),
    mesh=_sc_mesh,
    compiler_params=pltpu.CompilerParams(needs_layout_passes=False,
                                         use_tc_tiling_on_sc=False),
    scratch_types=[
        [pltpu.VMEM((PW,), jnp.int32)] * 2,
        [pltpu.VMEM((PW,), jnp.int32)] * 2,
        [pltpu.VMEM((PW,), jnp.int32)] * 2,
        [pltpu.VMEM((W, QW), jnp.float32)] * 2,
        [pltpu.VMEM((W, 256), jnp.float32)] * 2,
        [pltpu.VMEM((W, EW), jnp.float32)] * 2,
        [pltpu.VMEM((PW, AW), jnp.float32)] * 2,
        pltpu.VMEM_SHARED((N, AW), jnp.float32),
        [pltpu.SemaphoreType.DMA] * 2,
        [pltpu.SemaphoreType.DMA] * 2,
        [pltpu.SemaphoreType.DMA] * 2,
    ],
)
def _sc_edge(sd_h, e0_h, e1_h, q0_h, q1_h, kv0_h, kv1_h, z_h,
             out0, out1, isrc, idst, scidx, qrows, kvrows, erows, upd,
             shared, gsem, isem, ssem):
    c = lax.axis_index("c")
    s = lax.axis_index("s")

    @pl.when(s < NS - 1)
    def _():
        pltpu.sync_copy(z_h.at[pl.ds(s * 632, 632)],
                        shared.at[pl.ds(s * 632, 632)])

    @pl.when(s == NS - 1)
    def _():
        pltpu.sync_copy(z_h.at[pl.ds((NS - 1) * 632, N - (NS - 1) * 632)],
                        shared.at[pl.ds((NS - 1) * 632, N - (NS - 1) * 632)])

    zv = jnp.zeros((16,), jnp.float32)
    for b in range(2):
        for r in range(W, PW):
            for col in range(0, AW, 16):
                upd[b][r, pl.ds(col, 16)] = zv

    plsc.subcore_barrier()

    lane = lax.iota(jnp.int32, 16)
    px1 = jnp.bitwise_xor(lane, 1)
    px2 = jnp.bitwise_xor(lane, 2)
    px4 = jnp.bitwise_xor(lane, 4)
    px8 = jnp.bitwise_xor(lane, 8)

    def _perm(v, idx):
        return lax.gather(
            v, idx[:, None],
            lax.GatherDimensionNumbers(offset_dims=(),
                                       collapsed_slice_dims=(0,),
                                       start_index_map=(0,)),
            (1,),
            mode=lax.GatherScatterMode.PROMISE_IN_BOUNDS)

    def _allsum(v):
        v = v + _perm(v, px1)
        v = v + _perm(v, px2)
        v = v + _perm(v, px4)
        return v + _perm(v, px8)

    def run(q_h, kv_h, ev_h, out_h):
        def idx2_start(b, w):
            r = s * NWIN + w
            pltpu.async_copy(sd_h.at[r, 0], isrc[b], isem[b])
            pltpu.async_copy(sd_h.at[r, 1], idst[b], isem[b])

        def idx2_wait(b):
            pltpu.make_async_copy(sd_h.at[0, 0], isrc[b], isem[b]).wait()
            pltpu.make_async_copy(sd_h.at[0, 1], idst[b], isem[b]).wait()

        def gathers_start(b, w):
            base = s * EPS_ + w * W
            pltpu.async_copy(q_h.at[idst[b].at[pl.ds(0, W)]], qrows[b], gsem[b])
            pltpu.async_copy(kv_h.at[isrc[b].at[pl.ds(0, W)]], kvrows[b], gsem[b])
            pltpu.async_copy(ev_h.at[pl.ds(base, W)], erows[b], gsem[b])

        def gathers_wait(b):
            pltpu.make_async_copy(q_h.at[idst[b].at[pl.ds(0, W)]], qrows[b],
                                  gsem[b]).wait()
            pltpu.make_async_copy(kv_h.at[isrc[b].at[pl.ds(0, W)]], kvrows[b],
                                  gsem[b]).wait()
            pltpu.make_async_copy(ev_h.at[pl.ds(0, W)], erows[b], gsem[b]).wait()

        def scatter_start(b):
            pltpu.async_copy(upd[b], shared.at[scidx[b]], ssem[b], add=True)

        def scatter_wait(b):
            pltpu.make_async_copy(upd[b], shared.at[scidx[b]], ssem[b]).wait()

        def copy_idx(b):
            scidx[b][pl.ds(0, 16)] = idst[b][pl.ds(0, 16)]
            scidx[b][pl.ds(16, 16)] = idst[b][pl.ds(16, 16)]

        def compute(b):
            qr = qrows[b]
            kr = kvrows[b]
            er = erows[b]
            ur = upd[b]

            def edge(e, carry2):
                ea = er[e, pl.ds(0, 16)]
                ptail = jnp.zeros((16,), jnp.float32)
                for hh in range(4):
                    q0v = qr[e, pl.ds(hh * 32, 16)]
                    q1v = qr[e, pl.ds(hh * 32 + 16, 16)]
                    qev = qr[e, pl.ds(128 + hh * 16, 16)]
                    k0 = kr[e, pl.ds(hh * 64, 16)]
                    k1 = kr[e, pl.ds(hh * 64 + 16, 16)]
                    sv = q0v * k0 + q1v * k1 + qev * ea
                    pv = jnp.exp(_allsum(sv) * INV_SQRT_DH)
                    v0 = kr[e, pl.ds(hh * 64 + 32, 16)] + er[e, pl.ds(16 + hh * 32, 16)]
                    v1 = kr[e, pl.ds(hh * 64 + 48, 16)] + er[e, pl.ds(32 + hh * 32, 16)]
                    ur[e, pl.ds(hh * 32, 16)] = v0 * pv
                    ur[e, pl.ds(hh * 32 + 16, 16)] = v1 * pv
                    ptail = jnp.where(lane == hh, pv, ptail)
                ur[e, pl.ds(128, 16)] = ptail
                return carry2

            lax.fori_loop(0, W, edge, 0, unroll=12)

        idx2_start(0, 0)
        idx2_wait(0)
        gathers_start(0, 0)
        idx2_start(1, 1)

        def step(i, carry):
            wA = 2 * i
            wB = 2 * i + 1
            idx2_wait(1)
            gathers_start(1, wB)
            gathers_wait(0)

            @pl.when(i > 0)
            def _():
                scatter_wait(0)

            copy_idx(0)

            @pl.when(wA + 2 < NWIN)
            def _():
                idx2_start(0, wA + 2)

            compute(0)
            scatter_start(0)

            @pl.when(wA + 2 < NWIN)
            def _():
                idx2_wait(0)
                gathers_start(0, wA + 2)

            gathers_wait(1)

            @pl.when(i > 0)
            def _():
                scatter_wait(1)

            copy_idx(1)

            @pl.when(wB + 2 < NWIN)
            def _():
                idx2_start(1, wB + 2)

            compute(1)
            scatter_start(1)
            return carry

        lax.fori_loop(0, NWIN // 2, step, 0)
        scatter_wait(0)
        scatter_wait(1)
        plsc.subcore_barrier()

        @pl.when(s < NS - 1)
        def _():
            pltpu.sync_copy(shared.at[pl.ds(s * 632, 632)],
                            out_h.at[pl.ds(s * 632, 632)])

        @pl.when(s == NS - 1)
        def _():
            pltpu.sync_copy(shared.at[pl.ds((NS - 1) * 632, N - (NS - 1) * 632)],
                            out_h.at[pl.ds((NS - 1) * 632, N - (NS - 1) * 632)])

    @pl.when(c == 0)
    def _():
        run(q0_h, kv0_h, e0_h, out0)

    @pl.when(c == 1)
    def _():
        run(q1_h, kv1_h, e1_h, out1)


def _tc2_body(x_ref, t0_ref, t1_ref, modt_ref, Wp_ref, bp_ref,
              E8_ref, W1_ref, b1_ref, W2_ref, b2_ref, o_ref):
    t0 = t0_ref[...]
    t1 = t1_ref[...]
    av = jnp.concatenate([t0[:, 0:128], t1[:, 0:128]], axis=-1)
    p8 = jnp.concatenate([t0[:, 128:132], t1[:, 128:132]], axis=-1)
    r = 1.0 / (p8 + 1e-16)
    rexp = jnp.dot(r, E8_ref[...], preferred_element_type=jnp.float32)
    msg = av * rexp
    attn = (jnp.dot(msg, Wp_ref[...], preferred_element_type=jnp.float32)
            + bp_ref[...])
    modt = modt_ref[...]
    x1 = x_ref[...] + modt[:, 0:D] * attn
    mu = jnp.mean(x1, axis=-1, keepdims=True)
    var = jnp.mean((x1 - mu) * (x1 - mu), axis=-1, keepdims=True)
    ln = (x1 - mu) * lax.rsqrt(var + LN_EPS)
    h2 = ln * (1.0 + modt[:, 2 * D:3 * D]) + modt[:, D:2 * D]
    g = jax.nn.gelu(jnp.dot(h2, W1_ref[...], preferred_element_type=jnp.float32)
                    + b1_ref[...], approximate=True)
    mlp = jnp.dot(g, W2_ref[...], preferred_element_type=jnp.float32) + b2_ref[...]
    o_ref[...] = x1 + modt[:, 3 * D:4 * D] * mlp


def _tc2(x, t0, t1, modt, Wp, bp, E8, W1, b1, W2, b2):
    BN = 1000
    grid = (N // BN,)
    return pl.pallas_call(
        _tc2_body,
        grid=grid,
        in_specs=[
            pl.BlockSpec((BN, D), lambda i: (i, 0)),
            pl.BlockSpec((BN, AW), lambda i: (i, 0)),
            pl.BlockSpec((BN, AW), lambda i: (i, 0)),
            pl.BlockSpec((BN, 4 * D), lambda i: (i, 0)),
            pl.BlockSpec((D, D), lambda i: (0, 0)),
            pl.BlockSpec((1, D), lambda i: (0, 0)),
            pl.BlockSpec((H, D), lambda i: (0, 0)),
            pl.BlockSpec((D, FF), lambda i: (0, 0)),
            pl.BlockSpec((1, FF), lambda i: (0, 0)),
            pl.BlockSpec((FF, D), lambda i: (0, 0)),
            pl.BlockSpec((1, D), lambda i: (0, 0)),
        ],
        out_specs=pl.BlockSpec((BN, D), lambda i: (i, 0)),
        out_shape=jax.ShapeDtypeStruct((N, D), jnp.float32),
    )(x, t0, t1, modt, Wp, bp, E8, W1, b1, W2, b2)


def kernel(x, edge_index, edge_attr, c, Wq, bq, Wkv, bkv, Wp, bp,
           W1, b1, W2, b2, Wa, ba):
    f32 = jnp.float32
    Wkvh = Wkv[:D]
    Wkve = Wkv[D:]
    hs = jnp.arange(H)
    Wq2e = jnp.zeros((D, H * ED), f32)
    Wvecat = jnp.zeros((ED, D), f32)
    E8 = jnp.zeros((H, D), f32)
    for h in range(H):
        Ke = lax.dynamic_slice(Wkve, (0, h * 2 * DH), (ED, DH))
        Ve = lax.dynamic_slice(Wkve, (0, h * 2 * DH + DH), (ED, DH))
        Wq2e = lax.dynamic_update_slice(Wq2e, Ke.T, (h * DH, h * ED))
        Wvecat = lax.dynamic_update_slice(Wvecat, Ve, (0, h * DH))
        E8 = lax.dynamic_update_slice(E8, jnp.ones((1, DH), f32), (h, h * DH))

    ba2 = ba.reshape(1, 6 * D)
    bq2 = bq.reshape(1, D)
    bkv2 = bkv.reshape(1, 2 * D)
    bp2 = bp.reshape(1, D)
    b12 = b1.reshape(1, FF)
    b22 = b2.reshape(1, D)

    q, qe, nkv, modt = _tc1(x, c, Wa, ba2, Wq, bq2, Wkvh, bkv2, Wq2e)

    src = edge_index[0]
    dst = edge_index[1]
    e0, e1 = _tcev(edge_attr, Wvecat)
    zeros = jnp.zeros((N, AW), f32)
    qc0 = jnp.concatenate([q[:, :128], qe[:, :64]], axis=1)
    qc1 = jnp.concatenate([q[:, 128:], qe[:, 64:]], axis=1)
    sd = jnp.pad(edge_index.reshape(2, NS, NWIN, W),
                 ((0, 0), (0, 0), (0, 0), (0, PW - W)))
    sd = sd.transpose(1, 2, 0, 3).reshape(NS * NWIN, 2, PW)
    t0, t1 = _sc_edge(sd, e0, e1,
                      qc0, qc1,
                      nkv[:, :256], nkv[:, 256:],
                      zeros)

    return _tc2(x, t0, t1, modt, Wp, bp2, E8, W1, b12, W2, b22)


# speedup vs baseline: 1.7182x; 1.7182x over previous
import functools
import math

import jax
import jax.numpy as jnp
from jax import lax
from jax.experimental import pallas as pl
from jax.experimental.pallas import tpu as pltpu
from jax.experimental.pallas import tpu_sc as plsc

N = 10000
E = 160000
D = 256
H = 8
DH = 32
ED = 16
FF = 1024

NC = 2
NS = 16
LN_EPS = 1e-6
INV_SQRT_DH = 1.0 / math.sqrt(DH)

W = 25
PW = 32
EPS_ = E // NS
NWIN = EPS_ // W
AW = 144
QW = 192
EW = 144


def _tc1_body(x_ref, c_ref, Wa_ref, ba_ref, Wq_ref, bq_ref, Wkvh_ref, bkv_ref,
              Wq2e_ref, q_ref, qe_ref, nkv_ref, modt_ref):
    cc = c_ref[...]
    sc = cc * jax.nn.sigmoid(cc)
    mod = jnp.dot(sc, Wa_ref[...], preferred_element_type=jnp.float32) + ba_ref[...]
    xx = x_ref[...]
    mu = jnp.mean(xx, axis=-1, keepdims=True)
    var = jnp.mean((xx - mu) * (xx - mu), axis=-1, keepdims=True)
    ln = (xx - mu) * lax.rsqrt(var + LN_EPS)
    h = ln * (1.0 + mod[:, D:2 * D]) + mod[:, 0:D]
    q = jnp.dot(h, Wq_ref[...], preferred_element_type=jnp.float32) + bq_ref[...]
    q_ref[...] = q
    qe_ref[...] = jnp.dot(q, Wq2e_ref[...], preferred_element_type=jnp.float32)
    nkv_ref[...] = jnp.dot(h, Wkvh_ref[...], preferred_element_type=jnp.float32) + bkv_ref[...]
    modt_ref[...] = mod[:, 2 * D:6 * D]


def _tc1(x, c, Wa, ba, Wq, bq, Wkvh, bkv, Wq2e):
    BN = 1000
    grid = (N // BN,)
    return pl.pallas_call(
        _tc1_body,
        grid=grid,
        in_specs=[
            pl.BlockSpec((BN, D), lambda i: (i, 0)),
            pl.BlockSpec((BN, D), lambda i: (i, 0)),
            pl.BlockSpec((D, 6 * D), lambda i: (0, 0)),
            pl.BlockSpec((1, 6 * D), lambda i: (0, 0)),
            pl.BlockSpec((D, D), lambda i: (0, 0)),
            pl.BlockSpec((1, D), lambda i: (0, 0)),
            pl.BlockSpec((D, 2 * D), lambda i: (0, 0)),
            pl.BlockSpec((1, 2 * D), lambda i: (0, 0)),
            pl.BlockSpec((D, H * ED), lambda i: (0, 0)),
        ],
        out_specs=[
            pl.BlockSpec((BN, D), lambda i: (i, 0)),
            pl.BlockSpec((BN, H * ED), lambda i: (i, 0)),
            pl.BlockSpec((BN, 2 * D), lambda i: (i, 0)),
            pl.BlockSpec((BN, 4 * D), lambda i: (i, 0)),
        ],
        out_shape=[
            jax.ShapeDtypeStruct((N, D), jnp.float32),
            jax.ShapeDtypeStruct((N, H * ED), jnp.float32),
            jax.ShapeDtypeStruct((N, 2 * D), jnp.float32),
            jax.ShapeDtypeStruct((N, 4 * D), jnp.float32),
        ],
    )(x, c, Wa, ba, Wq, bq, Wkvh, bkv, Wq2e)


def _tcev_body(ea_ref, Wv_ref, e0_ref, e1_ref):
    ea = ea_ref[...]
    ev = jnp.dot(ea, Wv_ref[...], preferred_element_type=jnp.float32)
    e0_ref[...] = ev[:, :128].astype(jnp.bfloat16)
    e1_ref[...] = ev[:, 128:].astype(jnp.bfloat16)


def _tcev(edge_attr, Wvecat):
    BE = 4000
    return pl.pallas_call(
        _tcev_body,
        grid=(E // BE,),
        in_specs=[
            pl.BlockSpec((BE, ED), lambda i: (i, 0)),
            pl.BlockSpec((ED, D), lambda i: (0, 0)),
        ],
        out_specs=[pl.BlockSpec((BE, 128), lambda i: (i, 0)),
                   pl.BlockSpec((BE, 128), lambda i: (i, 0))],
        out_shape=[jax.ShapeDtypeStruct((E, 128), jnp.bfloat16),
                   jax.ShapeDtypeStruct((E, 128), jnp.bfloat16)],
    )(edge_attr, Wvecat)


_sc_mesh = plsc.VectorSubcoreMesh(
    core_axis_name="c", subcore_axis_name="s", num_cores=NC, num_subcores=NS)


@functools.partial(
    pl.kernel,
    out_type=(jax.ShapeDtypeStruct((N, AW), jnp.float32),
              jax.ShapeDtypeStruct((N, AW), jnp.float32)),
    mesh=_sc_mesh,
    compiler_params=pltpu.CompilerParams(needs_layout_passes=False,
                                         use_tc_tiling_on_sc=False),
    scratch_types=[
        [pltpu.VMEM((PW,), jnp.int32)] * 2,
        [pltpu.VMEM((PW,), jnp.int32)] * 2,
        [pltpu.VMEM((PW,), jnp.int32)] * 2,
        [pltpu.VMEM((W, 256), jnp.bfloat16)] * 2,
        [pltpu.VMEM((W, 256), jnp.bfloat16)] * 2,
        [pltpu.VMEM((W, 128), jnp.bfloat16)] * 2,
        [pltpu.VMEM((W, ED), jnp.float32)] * 2,
        [pltpu.VMEM((PW, AW), jnp.float32)] * 2,
        pltpu.VMEM_SHARED((N, AW), jnp.float32),
        [pltpu.SemaphoreType.DMA] * 2,
        [pltpu.SemaphoreType.DMA] * 2,
        [pltpu.SemaphoreType.DMA] * 2,
    ],
)
def _sc_edge(sd_h, ea_h, e0_h, e1_h, q0_h, q1_h, kv0_h, kv1_h, z_h,
             out0, out1, isrc, idst, scidx, qrows, kvrows, erows, earows, upd,
             shared, gsem, isem, ssem):
    c = lax.axis_index("c")
    s = lax.axis_index("s")

    @pl.when(s < NS - 1)
    def _():
        pltpu.sync_copy(z_h.at[pl.ds(s * 632, 632)],
                        shared.at[pl.ds(s * 632, 632)])

    @pl.when(s == NS - 1)
    def _():
        pltpu.sync_copy(z_h.at[pl.ds((NS - 1) * 632, N - (NS - 1) * 632)],
                        shared.at[pl.ds((NS - 1) * 632, N - (NS - 1) * 632)])

    zv = jnp.zeros((16,), jnp.float32)
    for b in range(2):
        for r in range(W, PW):
            for col in range(0, AW, 16):
                upd[b][r, pl.ds(col, 16)] = zv

    plsc.subcore_barrier()

    lane = lax.iota(jnp.int32, 16)
    px1 = jnp.bitwise_xor(lane, 1)
    px2 = jnp.bitwise_xor(lane, 2)
    px4 = jnp.bitwise_xor(lane, 4)
    px8 = jnp.bitwise_xor(lane, 8)

    def _perm(v, idx):
        return lax.gather(
            v, idx[:, None],
            lax.GatherDimensionNumbers(offset_dims=(),
                                       collapsed_slice_dims=(0,),
                                       start_index_map=(0,)),
            (1,),
            mode=lax.GatherScatterMode.PROMISE_IN_BOUNDS)

    def _allsum(v):
        v = v + _perm(v, px1)
        v = v + _perm(v, px2)
        v = v + _perm(v, px4)
        return v + _perm(v, px8)

    def run(q_h, kv_h, ev_h, out_h):
        def idx2_start(b, w):
            r = s * NWIN + w
            pltpu.async_copy(sd_h.at[r, 0], isrc[b], isem[b])
            pltpu.async_copy(sd_h.at[r, 1], idst[b], isem[b])

        def idx2_wait(b):
            pltpu.make_async_copy(sd_h.at[0, 0], isrc[b], isem[b]).wait()
            pltpu.make_async_copy(sd_h.at[0, 1], idst[b], isem[b]).wait()

        def gathers_start(b, w):
            base = s * EPS_ + w * W
            pltpu.async_copy(q_h.at[idst[b].at[pl.ds(0, W)]], qrows[b], gsem[b])
            pltpu.async_copy(kv_h.at[isrc[b].at[pl.ds(0, W)]], kvrows[b], gsem[b])
            pltpu.async_copy(ev_h.at[pl.ds(base, W)], erows[b], gsem[b])
            pltpu.async_copy(ea_h.at[pl.ds(base, W)], earows[b], gsem[b])

        def gathers_wait(b):
            pltpu.make_async_copy(q_h.at[idst[b].at[pl.ds(0, W)]], qrows[b],
                                  gsem[b]).wait()
            pltpu.make_async_copy(kv_h.at[isrc[b].at[pl.ds(0, W)]], kvrows[b],
                                  gsem[b]).wait()
            pltpu.make_async_copy(ev_h.at[pl.ds(0, W)], erows[b], gsem[b]).wait()
            pltpu.make_async_copy(ea_h.at[pl.ds(0, W)], earows[b], gsem[b]).wait()

        def scatter_start(b):
            pltpu.async_copy(upd[b], shared.at[scidx[b]], ssem[b], add=True)

        def scatter_wait(b):
            pltpu.make_async_copy(upd[b], shared.at[scidx[b]], ssem[b]).wait()

        def copy_idx(b):
            scidx[b][pl.ds(0, 16)] = idst[b][pl.ds(0, 16)]
            scidx[b][pl.ds(16, 16)] = idst[b][pl.ds(16, 16)]

        def compute(b):
            qr = qrows[b]
            kr = kvrows[b]
            er = erows[b]
            ar = earows[b]
            ur = upd[b]
            ilv = plsc.PackFormat.INTERLEAVED

            def edge(e, carry2):
                ea = ar[e, :]
                ptail = jnp.zeros((16,), jnp.float32)
                for hh in range(4):
                    q0v, q1v = plsc.unpack(qr[e, pl.ds(hh * 32, 32)], format=ilv)
                    qev = plsc.bitcast(qr[e, pl.ds(128 + hh * 32, 32)], jnp.float32)
                    k0, k1 = plsc.unpack(kr[e, pl.ds(hh * 64, 32)], format=ilv)
                    sv = q0v * k0 + q1v * k1 + qev * ea
                    pv = jnp.exp(_allsum(sv) * INV_SQRT_DH)
                    vn0, vn1 = plsc.unpack(kr[e, pl.ds(hh * 64 + 32, 32)], format=ilv)
                    ve0, ve1 = plsc.unpack(er[e, pl.ds(hh * 32, 32)], format=ilv)
                    ur[e, pl.ds(hh * 32, 16)] = (vn0 + ve0) * pv
                    ur[e, pl.ds(hh * 32 + 16, 16)] = (vn1 + ve1) * pv
                    ptail = jnp.where(lane == hh, pv, ptail)
                ur[e, pl.ds(128, 16)] = ptail
                return carry2

            lax.fori_loop(0, W, edge, 0, unroll=5)

        idx2_start(0, 0)
        idx2_wait(0)
        gathers_start(0, 0)
        idx2_start(1, 1)

        def step(i, carry):
            wA = 2 * i
            wB = 2 * i + 1
            idx2_wait(1)
            gathers_start(1, wB)
            gathers_wait(0)

            @pl.when(i > 0)
            def _():
                scatter_wait(0)

            copy_idx(0)

            @pl.when(wA + 2 < NWIN)
            def _():
                idx2_start(0, wA + 2)

            compute(0)
            scatter_start(0)

            @pl.when(wA + 2 < NWIN)
            def _():
                idx2_wait(0)
                gathers_start(0, wA + 2)

            gathers_wait(1)

            @pl.when(i > 0)
            def _():
                scatter_wait(1)

            copy_idx(1)

            @pl.when(wB + 2 < NWIN)
            def _():
                idx2_start(1, wB + 2)

            compute(1)
            scatter_start(1)
            return carry

        lax.fori_loop(0, NWIN // 2, step, 0)
        scatter_wait(0)
        scatter_wait(1)
        plsc.subcore_barrier()

        @pl.when(s < NS - 1)
        def _():
            pltpu.sync_copy(shared.at[pl.ds(s * 632, 632)],
                            out_h.at[pl.ds(s * 632, 632)])

        @pl.when(s == NS - 1)
        def _():
            pltpu.sync_copy(shared.at[pl.ds((NS - 1) * 632, N - (NS - 1) * 632)],
                            out_h.at[pl.ds((NS - 1) * 632, N - (NS - 1) * 632)])

    @pl.when(c == 0)
    def _():
        run(q0_h, kv0_h, e0_h, out0)

    @pl.when(c == 1)
    def _():
        run(q1_h, kv1_h, e1_h, out1)


def _tc2_body(x_ref, t0_ref, t1_ref, modt_ref, Wp_ref, bp_ref,
              E8_ref, W1_ref, b1_ref, W2_ref, b2_ref, o_ref):
    t0 = t0_ref[...]
    t1 = t1_ref[...]
    av = jnp.concatenate([t0[:, 0:128], t1[:, 0:128]], axis=-1)
    p8 = jnp.concatenate([t0[:, 128:132], t1[:, 128:132]], axis=-1)
    r = 1.0 / (p8 + 1e-16)
    rexp = jnp.dot(r, E8_ref[...], preferred_element_type=jnp.float32)
    msg = av * rexp
    attn = (jnp.dot(msg, Wp_ref[...], preferred_element_type=jnp.float32)
            + bp_ref[...])
    modt = modt_ref[...]
    x1 = x_ref[...] + modt[:, 0:D] * attn
    mu = jnp.mean(x1, axis=-1, keepdims=True)
    var = jnp.mean((x1 - mu) * (x1 - mu), axis=-1, keepdims=True)
    ln = (x1 - mu) * lax.rsqrt(var + LN_EPS)
    h2 = ln * (1.0 + modt[:, 2 * D:3 * D]) + modt[:, D:2 * D]
    g = jax.nn.gelu(jnp.dot(h2, W1_ref[...], preferred_element_type=jnp.float32)
                    + b1_ref[...], approximate=True)
    mlp = jnp.dot(g, W2_ref[...], preferred_element_type=jnp.float32) + b2_ref[...]
    o_ref[...] = x1 + modt[:, 3 * D:4 * D] * mlp


def _tc2(x, t0, t1, modt, Wp, bp, E8, W1, b1, W2, b2):
    BN = 1000
    grid = (N // BN,)
    return pl.pallas_call(
        _tc2_body,
        grid=grid,
        in_specs=[
            pl.BlockSpec((BN, D), lambda i: (i, 0)),
            pl.BlockSpec((BN, AW), lambda i: (i, 0)),
            pl.BlockSpec((BN, AW), lambda i: (i, 0)),
            pl.BlockSpec((BN, 4 * D), lambda i: (i, 0)),
            pl.BlockSpec((D, D), lambda i: (0, 0)),
            pl.BlockSpec((1, D), lambda i: (0, 0)),
            pl.BlockSpec((H, D), lambda i: (0, 0)),
            pl.BlockSpec((D, FF), lambda i: (0, 0)),
            pl.BlockSpec((1, FF), lambda i: (0, 0)),
            pl.BlockSpec((FF, D), lambda i: (0, 0)),
            pl.BlockSpec((1, D), lambda i: (0, 0)),
        ],
        out_specs=pl.BlockSpec((BN, D), lambda i: (i, 0)),
        out_shape=jax.ShapeDtypeStruct((N, D), jnp.float32),
    )(x, t0, t1, modt, Wp, bp, E8, W1, b1, W2, b2)


def kernel(x, edge_index, edge_attr, c, Wq, bq, Wkv, bkv, Wp, bp,
           W1, b1, W2, b2, Wa, ba):
    f32 = jnp.float32
    Wkvh = Wkv[:D]
    Wkve = Wkv[D:]
    hs = jnp.arange(H)
    Wq2e = jnp.zeros((D, H * ED), f32)
    Wvecat = jnp.zeros((ED, D), f32)
    E8 = jnp.zeros((H, D), f32)
    for h in range(H):
        Ke = lax.dynamic_slice(Wkve, (0, h * 2 * DH), (ED, DH))
        Ve = lax.dynamic_slice(Wkve, (0, h * 2 * DH + DH), (ED, DH))
        Wq2e = lax.dynamic_update_slice(Wq2e, Ke.T, (h * DH, h * ED))
        Wvecat = lax.dynamic_update_slice(Wvecat, Ve, (0, h * DH))
        E8 = lax.dynamic_update_slice(E8, jnp.ones((1, DH), f32), (h, h * DH))

    ba2 = ba.reshape(1, 6 * D)
    bq2 = bq.reshape(1, D)
    bkv2 = bkv.reshape(1, 2 * D)
    bp2 = bp.reshape(1, D)
    b12 = b1.reshape(1, FF)
    b22 = b2.reshape(1, D)

    q, qe, nkv, modt = _tc1(x, c, Wa, ba2, Wq, bq2, Wkvh, bkv2, Wq2e)

    src = edge_index[0]
    dst = edge_index[1]
    e0, e1 = _tcev(edge_attr, Wvecat)
    zeros = jnp.zeros((N, AW), f32)
    bf16 = jnp.bfloat16

    def _f32bits(a):
        return lax.bitcast_convert_type(a, bf16).reshape(a.shape[0], -1)

    qc0 = jnp.concatenate([q[:, :128].astype(bf16), _f32bits(qe[:, :64])], axis=1)
    qc1 = jnp.concatenate([q[:, 128:].astype(bf16), _f32bits(qe[:, 64:])], axis=1)
    sd = jnp.pad(edge_index.reshape(2, NS, NWIN, W),
                 ((0, 0), (0, 0), (0, 0), (0, PW - W)))
    sd = sd.transpose(1, 2, 0, 3).reshape(NS * NWIN, 2, PW)
    t0, t1 = _sc_edge(sd, edge_attr, e0, e1,
                      qc0, qc1,
                      nkv[:, :256].astype(bf16), nkv[:, 256:].astype(bf16),
                      zeros)

    perm = jnp.arange(H * DH).reshape(H, DH // 2, 2).transpose(0, 2, 1).reshape(-1)
    Wp_perm = Wp[perm]
    return _tc2(x, t0, t1, modt, Wp_perm, bp2, E8, W1, b12, W2, b22)


# speedup vs baseline: 1.7183x; 1.0001x over previous
import functools
import math

import jax
import jax.numpy as jnp
from jax import lax
from jax.experimental import pallas as pl
from jax.experimental.pallas import tpu as pltpu
from jax.experimental.pallas import tpu_sc as plsc

N = 10000
E = 160000
D = 256
H = 8
DH = 32
ED = 16
FF = 1024

NC = 2
NS = 16
LN_EPS = 1e-6
INV_SQRT_DH = 1.0 / math.sqrt(DH)

W = 25
PW = 32
EPS_ = E // NS
NWIN = EPS_ // W
AW = 144
QW = 192
EW = 144


def _tc1_body(x_ref, c_ref, Wa_ref, ba_ref, Wq_ref, bq_ref, Wkvh_ref, bkv_ref,
              Wq2e_ref, q_ref, qe_ref, nkv_ref, modt_ref):
    cc = c_ref[...]
    sc = cc * jax.nn.sigmoid(cc)
    mod = jnp.dot(sc, Wa_ref[...], preferred_element_type=jnp.float32) + ba_ref[...]
    xx = x_ref[...]
    mu = jnp.mean(xx, axis=-1, keepdims=True)
    var = jnp.mean((xx - mu) * (xx - mu), axis=-1, keepdims=True)
    ln = (xx - mu) * lax.rsqrt(var + LN_EPS)
    h = ln * (1.0 + mod[:, D:2 * D]) + mod[:, 0:D]
    q = jnp.dot(h, Wq_ref[...], preferred_element_type=jnp.float32) + bq_ref[...]
    q_ref[...] = q
    qe_ref[...] = jnp.dot(q, Wq2e_ref[...], preferred_element_type=jnp.float32)
    nkv_ref[...] = jnp.dot(h, Wkvh_ref[...], preferred_element_type=jnp.float32) + bkv_ref[...]
    modt_ref[...] = mod[:, 2 * D:6 * D]


def _tc1(x, c, Wa, ba, Wq, bq, Wkvh, bkv, Wq2e):
    BN = 1000
    grid = (N // BN,)
    return pl.pallas_call(
        _tc1_body,
        grid=grid,
        in_specs=[
            pl.BlockSpec((BN, D), lambda i: (i, 0)),
            pl.BlockSpec((BN, D), lambda i: (i, 0)),
            pl.BlockSpec((D, 6 * D), lambda i: (0, 0)),
            pl.BlockSpec((1, 6 * D), lambda i: (0, 0)),
            pl.BlockSpec((D, D), lambda i: (0, 0)),
            pl.BlockSpec((1, D), lambda i: (0, 0)),
            pl.BlockSpec((D, 2 * D), lambda i: (0, 0)),
            pl.BlockSpec((1, 2 * D), lambda i: (0, 0)),
            pl.BlockSpec((D, H * ED), lambda i: (0, 0)),
        ],
        out_specs=[
            pl.BlockSpec((BN, D), lambda i: (i, 0)),
            pl.BlockSpec((BN, H * ED), lambda i: (i, 0)),
            pl.BlockSpec((BN, 2 * D), lambda i: (i, 0)),
            pl.BlockSpec((BN, 4 * D), lambda i: (i, 0)),
        ],
        out_shape=[
            jax.ShapeDtypeStruct((N, D), jnp.float32),
            jax.ShapeDtypeStruct((N, H * ED), jnp.float32),
            jax.ShapeDtypeStruct((N, 2 * D), jnp.float32),
            jax.ShapeDtypeStruct((N, 4 * D), jnp.float32),
        ],
    )(x, c, Wa, ba, Wq, bq, Wkvh, bkv, Wq2e)


def _tcev_body(ea_ref, Wv_ref, e0_ref, e1_ref):
    ea = ea_ref[...]
    ev = jnp.dot(ea, Wv_ref[...], preferred_element_type=jnp.float32)
    e0_ref[...] = ev[:, :128].astype(jnp.bfloat16)
    e1_ref[...] = ev[:, 128:].astype(jnp.bfloat16)


def _tcev(edge_attr, Wvecat):
    BE = 4000
    return pl.pallas_call(
        _tcev_body,
        grid=(E // BE,),
        in_specs=[
            pl.BlockSpec((BE, ED), lambda i: (i, 0)),
            pl.BlockSpec((ED, D), lambda i: (0, 0)),
        ],
        out_specs=[pl.BlockSpec((BE, 128), lambda i: (i, 0)),
                   pl.BlockSpec((BE, 128), lambda i: (i, 0))],
        out_shape=[jax.ShapeDtypeStruct((E, 128), jnp.bfloat16),
                   jax.ShapeDtypeStruct((E, 128), jnp.bfloat16)],
    )(edge_attr, Wvecat)


_sc_mesh = plsc.VectorSubcoreMesh(
    core_axis_name="c", subcore_axis_name="s", num_cores=NC, num_subcores=NS)


@functools.partial(
    pl.kernel,
    out_type=(jax.ShapeDtypeStruct((N, AW), jnp.float32),
              jax.ShapeDtypeStruct((N, AW), jnp.float32)),
    mesh=_sc_mesh,
    compiler_params=pltpu.CompilerParams(needs_layout_passes=False,
                                         use_tc_tiling_on_sc=False),
    scratch_types=[
        [pltpu.VMEM((PW,), jnp.int32)] * 2,
        [pltpu.VMEM((PW,), jnp.int32)] * 2,
        [pltpu.VMEM((PW,), jnp.int32)] * 2,
        [pltpu.VMEM((W, 256), jnp.bfloat16)] * 2,
        [pltpu.VMEM((W, 256), jnp.bfloat16)] * 2,
        [pltpu.VMEM((W, 128), jnp.bfloat16)] * 2,
        [pltpu.VMEM((W, ED), jnp.float32)] * 2,
        [pltpu.VMEM((PW, AW), jnp.float32)] * 2,
        pltpu.VMEM_SHARED((N, AW), jnp.float32),
        [pltpu.SemaphoreType.DMA] * 2,
        [pltpu.SemaphoreType.DMA] * 2,
        [pltpu.SemaphoreType.DMA] * 2,
    ],
)
def _sc_edge(sd_h, ea_h, e0_h, e1_h, q0_h, q1_h, kv0_h, kv1_h, z_h,
             out0, out1, isrc, idst, scidx, qrows, kvrows, erows, earows, upd,
             shared, gsem, isem, ssem):
    c = lax.axis_index("c")
    s = lax.axis_index("s")

    @pl.when(s < NS - 1)
    def _():
        pltpu.sync_copy(z_h.at[pl.ds(s * 632, 632)],
                        shared.at[pl.ds(s * 632, 632)])

    @pl.when(s == NS - 1)
    def _():
        pltpu.sync_copy(z_h.at[pl.ds((NS - 1) * 632, N - (NS - 1) * 632)],
                        shared.at[pl.ds((NS - 1) * 632, N - (NS - 1) * 632)])

    zv = jnp.zeros((16,), jnp.float32)
    for b in range(2):
        for r in range(W, PW):
            for col in range(0, AW, 16):
                upd[b][r, pl.ds(col, 16)] = zv

    plsc.subcore_barrier()

    lane = lax.iota(jnp.int32, 16)
    px1 = jnp.bitwise_xor(lane, 1)
    px2 = jnp.bitwise_xor(lane, 2)
    px4 = jnp.bitwise_xor(lane, 4)
    px8 = jnp.bitwise_xor(lane, 8)

    def _perm(v, idx):
        return lax.gather(
            v, idx[:, None],
            lax.GatherDimensionNumbers(offset_dims=(),
                                       collapsed_slice_dims=(0,),
                                       start_index_map=(0,)),
            (1,),
            mode=lax.GatherScatterMode.PROMISE_IN_BOUNDS)

    def _allsum(v):
        v = v + _perm(v, px1)
        v = v + _perm(v, px2)
        v = v + _perm(v, px4)
        return v + _perm(v, px8)

    def run(q_h, kv_h, ev_h, out_h):
        def idx2_start(b, w):
            r = s * NWIN + w
            pltpu.async_copy(sd_h.at[r, 0], isrc[b], isem[b])
            pltpu.async_copy(sd_h.at[r, 1], idst[b], isem[b])

        def idx2_wait(b):
            pltpu.make_async_copy(sd_h.at[0, 0], isrc[b], isem[b]).wait()
            pltpu.make_async_copy(sd_h.at[0, 1], idst[b], isem[b]).wait()

        def gathers_start(b, w):
            base = s * EPS_ + w * W
            pltpu.async_copy(q_h.at[idst[b].at[pl.ds(0, W)]], qrows[b], gsem[b])
            pltpu.async_copy(kv_h.at[isrc[b].at[pl.ds(0, W)]], kvrows[b], gsem[b])
            pltpu.async_copy(ev_h.at[pl.ds(base, W)], erows[b], gsem[b])
            pltpu.async_copy(ea_h.at[pl.ds(base, W)], earows[b], gsem[b])

        def gathers_wait(b):
            pltpu.make_async_copy(q_h.at[idst[b].at[pl.ds(0, W)]], qrows[b],
                                  gsem[b]).wait()
            pltpu.make_async_copy(kv_h.at[isrc[b].at[pl.ds(0, W)]], kvrows[b],
                                  gsem[b]).wait()
            pltpu.make_async_copy(ev_h.at[pl.ds(0, W)], erows[b], gsem[b]).wait()
            pltpu.make_async_copy(ea_h.at[pl.ds(0, W)], earows[b], gsem[b]).wait()

        def scatter_start(b):
            pltpu.async_copy(upd[b], shared.at[scidx[b]], ssem[b], add=True)

        def scatter_wait(b):
            pltpu.make_async_copy(upd[b], shared.at[scidx[b]], ssem[b]).wait()

        def copy_idx(b):
            scidx[b][pl.ds(0, 16)] = idst[b][pl.ds(0, 16)]
            scidx[b][pl.ds(16, 16)] = idst[b][pl.ds(16, 16)]

        def compute(b):
            qr = qrows[b]
            kr = kvrows[b]
            er = erows[b]
            ar = earows[b]
            ur = upd[b]
            ilv = plsc.PackFormat.INTERLEAVED

            def edge(e, carry2):
                ea = ar[e, :]
                ptail = jnp.zeros((16,), jnp.float32)
                for hh in range(4):
                    qk = qr[e, pl.ds(hh * 32, 32)] * kr[e, pl.ds(hh * 64, 32)]
                    qev = plsc.bitcast(qr[e, pl.ds(128 + hh * 32, 32)], jnp.float32)
                    s0, s1 = plsc.unpack(qk, format=ilv)
                    sv = s0 + s1 + qev * ea
                    pv = jnp.exp(_allsum(sv) * INV_SQRT_DH)
                    vt = kr[e, pl.ds(hh * 64 + 32, 32)] + er[e, pl.ds(hh * 32, 32)]
                    v0, v1 = plsc.unpack(vt, format=ilv)
                    ur[e, pl.ds(hh * 32, 16)] = v0 * pv
                    ur[e, pl.ds(hh * 32 + 16, 16)] = v1 * pv
                    ptail = jnp.where(lane == hh, pv, ptail)
                ur[e, pl.ds(128, 16)] = ptail
                return carry2

            lax.fori_loop(0, W, edge, 0, unroll=5)

        idx2_start(0, 0)
        idx2_wait(0)
        gathers_start(0, 0)
        idx2_start(1, 1)

        def step(i, carry):
            wA = 2 * i
            wB = 2 * i + 1
            idx2_wait(1)
            gathers_start(1, wB)
            gathers_wait(0)

            @pl.when(i > 0)
            def _():
                scatter_wait(0)

            copy_idx(0)

            @pl.when(wA + 2 < NWIN)
            def _():
                idx2_start(0, wA + 2)

            compute(0)
            scatter_start(0)

            @pl.when(wA + 2 < NWIN)
            def _():
                idx2_wait(0)
                gathers_start(0, wA + 2)

            gathers_wait(1)

            @pl.when(i > 0)
            def _():
                scatter_wait(1)

            copy_idx(1)

            @pl.when(wB + 2 < NWIN)
            def _():
                idx2_start(1, wB + 2)

            compute(1)
            scatter_start(1)
            return carry

        lax.fori_loop(0, NWIN // 2, step, 0)
        scatter_wait(0)
        scatter_wait(1)
        plsc.subcore_barrier()

        @pl.when(s < NS - 1)
        def _():
            pltpu.sync_copy(shared.at[pl.ds(s * 632, 632)],
                            out_h.at[pl.ds(s * 632, 632)])

        @pl.when(s == NS - 1)
        def _():
            pltpu.sync_copy(shared.at[pl.ds((NS - 1) * 632, N - (NS - 1) * 632)],
                            out_h.at[pl.ds((NS - 1) * 632, N - (NS - 1) * 632)])

    @pl.when(c == 0)
    def _():
        run(q0_h, kv0_h, e0_h, out0)

    @pl.when(c == 1)
    def _():
        run(q1_h, kv1_h, e1_h, out1)


def _tc2_body(x_ref, t0_ref, t1_ref, modt_ref, Wp_ref, bp_ref,
              E8_ref, W1_ref, b1_ref, W2_ref, b2_ref, o_ref):
    t0 = t0_ref[...]
    t1 = t1_ref[...]
    av = jnp.concatenate([t0[:, 0:128], t1[:, 0:128]], axis=-1)
    p8 = jnp.concatenate([t0[:, 128:132], t1[:, 128:132]], axis=-1)
    r = 1.0 / (p8 + 1e-16)
    rexp = jnp.dot(r, E8_ref[...], preferred_element_type=jnp.float32)
    msg = av * rexp
    attn = (jnp.dot(msg, Wp_ref[...], preferred_element_type=jnp.float32)
            + bp_ref[...])
    modt = modt_ref[...]
    x1 = x_ref[...] + modt[:, 0:D] * attn
    mu = jnp.mean(x1, axis=-1, keepdims=True)
    var = jnp.mean((x1 - mu) * (x1 - mu), axis=-1, keepdims=True)
    ln = (x1 - mu) * lax.rsqrt(var + LN_EPS)
    h2 = ln * (1.0 + modt[:, 2 * D:3 * D]) + modt[:, D:2 * D]
    g = jax.nn.gelu(jnp.dot(h2, W1_ref[...], preferred_element_type=jnp.float32)
                    + b1_ref[...], approximate=True)
    mlp = jnp.dot(g, W2_ref[...], preferred_element_type=jnp.float32) + b2_ref[...]
    o_ref[...] = x1 + modt[:, 3 * D:4 * D] * mlp


def _tc2(x, t0, t1, modt, Wp, bp, E8, W1, b1, W2, b2):
    BN = 1000
    grid = (N // BN,)
    return pl.pallas_call(
        _tc2_body,
        grid=grid,
        in_specs=[
            pl.BlockSpec((BN, D), lambda i: (i, 0)),
            pl.BlockSpec((BN, AW), lambda i: (i, 0)),
            pl.BlockSpec((BN, AW), lambda i: (i, 0)),
            pl.BlockSpec((BN, 4 * D), lambda i: (i, 0)),
            pl.BlockSpec((D, D), lambda i: (0, 0)),
            pl.BlockSpec((1, D), lambda i: (0, 0)),
            pl.BlockSpec((H, D), lambda i: (0, 0)),
            pl.BlockSpec((D, FF), lambda i: (0, 0)),
            pl.BlockSpec((1, FF), lambda i: (0, 0)),
            pl.BlockSpec((FF, D), lambda i: (0, 0)),
            pl.BlockSpec((1, D), lambda i: (0, 0)),
        ],
        out_specs=pl.BlockSpec((BN, D), lambda i: (i, 0)),
        out_shape=jax.ShapeDtypeStruct((N, D), jnp.float32),
    )(x, t0, t1, modt, Wp, bp, E8, W1, b1, W2, b2)


def kernel(x, edge_index, edge_attr, c, Wq, bq, Wkv, bkv, Wp, bp,
           W1, b1, W2, b2, Wa, ba):
    f32 = jnp.float32
    Wkvh = Wkv[:D]
    Wkve = Wkv[D:]
    hs = jnp.arange(H)
    Wq2e = jnp.zeros((D, H * ED), f32)
    Wvecat = jnp.zeros((ED, D), f32)
    E8 = jnp.zeros((H, D), f32)
    for h in range(H):
        Ke = lax.dynamic_slice(Wkve, (0, h * 2 * DH), (ED, DH))
        Ve = lax.dynamic_slice(Wkve, (0, h * 2 * DH + DH), (ED, DH))
        Wq2e = lax.dynamic_update_slice(Wq2e, Ke.T, (h * DH, h * ED))
        Wvecat = lax.dynamic_update_slice(Wvecat, Ve, (0, h * DH))
        E8 = lax.dynamic_update_slice(E8, jnp.ones((1, DH), f32), (h, h * DH))

    ba2 = ba.reshape(1, 6 * D)
    bq2 = bq.reshape(1, D)
    bkv2 = bkv.reshape(1, 2 * D)
    bp2 = bp.reshape(1, D)
    b12 = b1.reshape(1, FF)
    b22 = b2.reshape(1, D)

    q, qe, nkv, modt = _tc1(x, c, Wa, ba2, Wq, bq2, Wkvh, bkv2, Wq2e)

    src = edge_index[0]
    dst = edge_index[1]
    e0, e1 = _tcev(edge_attr, Wvecat)
    zeros = jnp.zeros((N, AW), f32)
    bf16 = jnp.bfloat16

    def _f32bits(a):
        return lax.bitcast_convert_type(a, bf16).reshape(a.shape[0], -1)

    qc0 = jnp.concatenate([q[:, :128].astype(bf16), _f32bits(qe[:, :64])], axis=1)
    qc1 = jnp.concatenate([q[:, 128:].astype(bf16), _f32bits(qe[:, 64:])], axis=1)
    sd = jnp.pad(edge_index.reshape(2, NS, NWIN, W),
                 ((0, 0), (0, 0), (0, 0), (0, PW - W)))
    sd = sd.transpose(1, 2, 0, 3).reshape(NS * NWIN, 2, PW)
    t0, t1 = _sc_edge(sd, edge_attr, e0, e1,
                      qc0, qc1,
                      nkv[:, :256].astype(bf16), nkv[:, 256:].astype(bf16),
                      zeros)

    perm = jnp.arange(H * DH).reshape(H, DH // 2, 2).transpose(0, 2, 1).reshape(-1)
    Wp_perm = Wp[perm]
    return _tc2(x, t0, t1, modt, Wp_perm, bp2, E8, W1, b12, W2, b22)


# speedup vs baseline: 1.7496x; 1.0182x over previous
import functools
import math

import jax
import jax.numpy as jnp
from jax import lax
from jax.experimental import pallas as pl
from jax.experimental.pallas import tpu as pltpu
from jax.experimental.pallas import tpu_sc as plsc

N = 10000
E = 160000
D = 256
H = 8
DH = 32
ED = 16
FF = 1024

NC = 2
NS = 16
LN_EPS = 1e-6
INV_SQRT_DH = 1.0 / math.sqrt(DH)

W = 25
PW = 32
EPS_ = E // NS
NWIN = EPS_ // W
AW = 144
QW = 192
EW = 144


def _tc1_body(x_ref, c_ref, Wa_ref, ba_ref, Wq_ref, bq_ref, Wkvh_ref, bkv_ref,
              Wq2e_ref, q_ref, qe_ref, nkv_ref, modt_ref):
    cc = c_ref[...]
    sc = cc * jax.nn.sigmoid(cc)
    mod = jnp.dot(sc, Wa_ref[...], preferred_element_type=jnp.float32) + ba_ref[...]
    xx = x_ref[...]
    mu = jnp.mean(xx, axis=-1, keepdims=True)
    var = jnp.mean((xx - mu) * (xx - mu), axis=-1, keepdims=True)
    ln = (xx - mu) * lax.rsqrt(var + LN_EPS)
    h = ln * (1.0 + mod[:, D:2 * D]) + mod[:, 0:D]
    q = jnp.dot(h, Wq_ref[...], preferred_element_type=jnp.float32) + bq_ref[...]
    q_ref[...] = q
    qe_ref[...] = jnp.dot(q, Wq2e_ref[...], preferred_element_type=jnp.float32)
    nkv_ref[...] = jnp.dot(h, Wkvh_ref[...], preferred_element_type=jnp.float32) + bkv_ref[...]
    modt_ref[...] = mod[:, 2 * D:6 * D]


def _tc1(x, c, Wa, ba, Wq, bq, Wkvh, bkv, Wq2e):
    BN = 1000
    grid = (N // BN,)
    return pl.pallas_call(
        _tc1_body,
        grid=grid,
        in_specs=[
            pl.BlockSpec((BN, D), lambda i: (i, 0)),
            pl.BlockSpec((BN, D), lambda i: (i, 0)),
            pl.BlockSpec((D, 6 * D), lambda i: (0, 0)),
            pl.BlockSpec((1, 6 * D), lambda i: (0, 0)),
            pl.BlockSpec((D, D), lambda i: (0, 0)),
            pl.BlockSpec((1, D), lambda i: (0, 0)),
            pl.BlockSpec((D, 2 * D), lambda i: (0, 0)),
            pl.BlockSpec((1, 2 * D), lambda i: (0, 0)),
            pl.BlockSpec((D, H * ED), lambda i: (0, 0)),
        ],
        out_specs=[
            pl.BlockSpec((BN, D), lambda i: (i, 0)),
            pl.BlockSpec((BN, H * ED), lambda i: (i, 0)),
            pl.BlockSpec((BN, 2 * D), lambda i: (i, 0)),
            pl.BlockSpec((BN, 4 * D), lambda i: (i, 0)),
        ],
        out_shape=[
            jax.ShapeDtypeStruct((N, D), jnp.float32),
            jax.ShapeDtypeStruct((N, H * ED), jnp.float32),
            jax.ShapeDtypeStruct((N, 2 * D), jnp.float32),
            jax.ShapeDtypeStruct((N, 4 * D), jnp.float32),
        ],
    )(x, c, Wa, ba, Wq, bq, Wkvh, bkv, Wq2e)


def _tcev_body(ea_ref, Wv_ref, e0_ref, e1_ref):
    ea = ea_ref[...]
    ev = jnp.dot(ea, Wv_ref[...], preferred_element_type=jnp.float32)
    e0_ref[...] = ev[:, :128].astype(jnp.bfloat16)
    e1_ref[...] = ev[:, 128:].astype(jnp.bfloat16)


def _tcev(edge_attr, Wvecat):
    BE = 4000
    return pl.pallas_call(
        _tcev_body,
        grid=(E // BE,),
        in_specs=[
            pl.BlockSpec((BE, ED), lambda i: (i, 0)),
            pl.BlockSpec((ED, D), lambda i: (0, 0)),
        ],
        out_specs=[pl.BlockSpec((BE, 128), lambda i: (i, 0)),
                   pl.BlockSpec((BE, 128), lambda i: (i, 0))],
        out_shape=[jax.ShapeDtypeStruct((E, 128), jnp.bfloat16),
                   jax.ShapeDtypeStruct((E, 128), jnp.bfloat16)],
    )(edge_attr, Wvecat)


_sc_mesh = plsc.VectorSubcoreMesh(
    core_axis_name="c", subcore_axis_name="s", num_cores=NC, num_subcores=NS)


@functools.partial(
    pl.kernel,
    out_type=(jax.ShapeDtypeStruct((N, AW), jnp.float32),
              jax.ShapeDtypeStruct((N, AW), jnp.float32)),
    mesh=_sc_mesh,
    compiler_params=pltpu.CompilerParams(needs_layout_passes=False,
                                         use_tc_tiling_on_sc=False),
    scratch_types=[
        [pltpu.VMEM((PW,), jnp.int32)] * 2,
        [pltpu.VMEM((PW,), jnp.int32)] * 2,
        [pltpu.VMEM((PW,), jnp.int32)] * 2,
        [pltpu.VMEM((W, 256), jnp.bfloat16)] * 2,
        [pltpu.VMEM((W, 256), jnp.bfloat16)] * 2,
        [pltpu.VMEM((W, 128), jnp.bfloat16)] * 2,
        [pltpu.VMEM((W, ED), jnp.float32)] * 2,
        [pltpu.VMEM((PW, AW), jnp.float32)] * 2,
        pltpu.VMEM_SHARED((N, AW), jnp.float32),
        [pltpu.SemaphoreType.DMA] * 2,
        [pltpu.SemaphoreType.DMA] * 2,
        [pltpu.SemaphoreType.DMA] * 2,
    ],
)
def _sc_edge(sd_h, ea_h, e0_h, e1_h, q0_h, q1_h, kv0_h, kv1_h, z_h,
             out0, out1, isrc, idst, scidx, qrows, kvrows, erows, earows, upd,
             shared, gsem, isem, ssem):
    c = lax.axis_index("c")
    s = lax.axis_index("s")

    @pl.when(s < NS - 1)
    def _():
        pltpu.sync_copy(z_h.at[pl.ds(s * 632, 632)],
                        shared.at[pl.ds(s * 632, 632)])

    @pl.when(s == NS - 1)
    def _():
        pltpu.sync_copy(z_h.at[pl.ds((NS - 1) * 632, N - (NS - 1) * 632)],
                        shared.at[pl.ds((NS - 1) * 632, N - (NS - 1) * 632)])

    zv = jnp.zeros((16,), jnp.float32)
    for b in range(2):
        for r in range(W, PW):
            for col in range(0, AW, 16):
                upd[b][r, pl.ds(col, 16)] = zv

    plsc.subcore_barrier()

    lane = lax.iota(jnp.int32, 16)
    px1 = jnp.bitwise_xor(lane, 1)
    px2 = jnp.bitwise_xor(lane, 2)
    px4 = jnp.bitwise_xor(lane, 4)
    px8 = jnp.bitwise_xor(lane, 8)

    def _perm(v, idx):
        return lax.gather(
            v, idx[:, None],
            lax.GatherDimensionNumbers(offset_dims=(),
                                       collapsed_slice_dims=(0,),
                                       start_index_map=(0,)),
            (1,),
            mode=lax.GatherScatterMode.PROMISE_IN_BOUNDS)

    def _allsum(v):
        v = v + _perm(v, px1)
        v = v + _perm(v, px2)
        v = v + _perm(v, px4)
        return v + _perm(v, px8)

    def run(q_h, kv_h, ev_h, out_h):
        def idx2_start(b, w):
            r = s * NWIN + w
            pltpu.async_copy(sd_h.at[r, 0], isrc[b], isem[b])
            pltpu.async_copy(sd_h.at[r, 1], idst[b], isem[b])

        def idx2_wait(b):
            pltpu.make_async_copy(sd_h.at[0, 0], isrc[b], isem[b]).wait()
            pltpu.make_async_copy(sd_h.at[0, 1], idst[b], isem[b]).wait()

        def gathers_start(b, w):
            base = s * EPS_ + w * W
            pltpu.async_copy(q_h.at[idst[b].at[pl.ds(0, W)]], qrows[b], gsem[b])
            pltpu.async_copy(kv_h.at[isrc[b].at[pl.ds(0, W)]], kvrows[b], gsem[b])
            pltpu.async_copy(ev_h.at[pl.ds(base, W)], erows[b], gsem[b])
            pltpu.async_copy(ea_h.at[pl.ds(base, W)], earows[b], gsem[b])

        def gathers_wait(b):
            pltpu.make_async_copy(q_h.at[idst[b].at[pl.ds(0, W)]], qrows[b],
                                  gsem[b]).wait()
            pltpu.make_async_copy(kv_h.at[isrc[b].at[pl.ds(0, W)]], kvrows[b],
                                  gsem[b]).wait()
            pltpu.make_async_copy(ev_h.at[pl.ds(0, W)], erows[b], gsem[b]).wait()
            pltpu.make_async_copy(ea_h.at[pl.ds(0, W)], earows[b], gsem[b]).wait()

        def scatter_start(b):
            pltpu.async_copy(upd[b], shared.at[scidx[b]], ssem[b], add=True)

        def scatter_wait(b):
            pltpu.make_async_copy(upd[b], shared.at[scidx[b]], ssem[b]).wait()

        def copy_idx(b):
            scidx[b][pl.ds(0, 16)] = idst[b][pl.ds(0, 16)]
            scidx[b][pl.ds(16, 16)] = idst[b][pl.ds(16, 16)]

        def compute(b):
            qr = qrows[b]
            kr = kvrows[b]
            er = erows[b]
            ar = earows[b]
            ur = upd[b]
            ilv = plsc.PackFormat.INTERLEAVED

            def edge(e, carry2):
                ea = ar[e, :]
                ptail = jnp.zeros((16,), jnp.float32)
                for hh in range(4):
                    qk = qr[e, pl.ds(hh * 32, 32)] * kr[e, pl.ds(hh * 64, 32)]
                    qev = plsc.bitcast(qr[e, pl.ds(128 + hh * 32, 32)], jnp.float32)
                    s0, s1 = plsc.unpack(qk, format=ilv)
                    sv = s0 + s1 + qev * ea
                    pv = jnp.exp(_allsum(sv) * INV_SQRT_DH)
                    vt = kr[e, pl.ds(hh * 64 + 32, 32)] + er[e, pl.ds(hh * 32, 32)]
                    v0, v1 = plsc.unpack(vt, format=ilv)
                    ur[e, pl.ds(hh * 32, 16)] = v0 * pv
                    ur[e, pl.ds(hh * 32 + 16, 16)] = v1 * pv
                    ptail = jnp.where(lane == hh, pv, ptail)
                ur[e, pl.ds(128, 16)] = ptail
                return carry2

            lax.fori_loop(0, W, edge, 0, unroll=2)

        idx2_start(0, 0)
        idx2_wait(0)
        gathers_start(0, 0)
        idx2_start(1, 1)

        def step(i, carry):
            wA = 2 * i
            wB = 2 * i + 1
            idx2_wait(1)
            gathers_start(1, wB)
            gathers_wait(0)

            @pl.when(i > 0)
            def _():
                scatter_wait(0)

            copy_idx(0)

            @pl.when(wA + 2 < NWIN)
            def _():
                idx2_start(0, wA + 2)

            compute(0)
            scatter_start(0)

            @pl.when(wA + 2 < NWIN)
            def _():
                idx2_wait(0)
                gathers_start(0, wA + 2)

            gathers_wait(1)

            @pl.when(i > 0)
            def _():
                scatter_wait(1)

            copy_idx(1)

            @pl.when(wB + 2 < NWIN)
            def _():
                idx2_start(1, wB + 2)

            compute(1)
            scatter_start(1)
            return carry

        lax.fori_loop(0, NWIN // 2, step, 0)
        scatter_wait(0)
        scatter_wait(1)
        plsc.subcore_barrier()

        @pl.when(s < NS - 1)
        def _():
            pltpu.sync_copy(shared.at[pl.ds(s * 632, 632)],
                            out_h.at[pl.ds(s * 632, 632)])

        @pl.when(s == NS - 1)
        def _():
            pltpu.sync_copy(shared.at[pl.ds((NS - 1) * 632, N - (NS - 1) * 632)],
                            out_h.at[pl.ds((NS - 1) * 632, N - (NS - 1) * 632)])

    @pl.when(c == 0)
    def _():
        run(q0_h, kv0_h, e0_h, out0)

    @pl.when(c == 1)
    def _():
        run(q1_h, kv1_h, e1_h, out1)


def _tc2_body(x_ref, t0_ref, t1_ref, modt_ref, Wp_ref, bp_ref,
              E8_ref, W1_ref, b1_ref, W2_ref, b2_ref, o_ref):
    t0 = t0_ref[...]
    t1 = t1_ref[...]
    av = jnp.concatenate([t0[:, 0:128], t1[:, 0:128]], axis=-1)
    p8 = jnp.concatenate([t0[:, 128:132], t1[:, 128:132]], axis=-1)
    r = 1.0 / (p8 + 1e-16)
    rexp = jnp.dot(r, E8_ref[...], preferred_element_type=jnp.float32)
    msg = av * rexp
    attn = (jnp.dot(msg, Wp_ref[...], preferred_element_type=jnp.float32)
            + bp_ref[...])
    modt = modt_ref[...]
    x1 = x_ref[...] + modt[:, 0:D] * attn
    mu = jnp.mean(x1, axis=-1, keepdims=True)
    var = jnp.mean((x1 - mu) * (x1 - mu), axis=-1, keepdims=True)
    ln = (x1 - mu) * lax.rsqrt(var + LN_EPS)
    h2 = ln * (1.0 + modt[:, 2 * D:3 * D]) + modt[:, D:2 * D]
    g = jax.nn.gelu(jnp.dot(h2, W1_ref[...], preferred_element_type=jnp.float32)
                    + b1_ref[...], approximate=True)
    mlp = jnp.dot(g, W2_ref[...], preferred_element_type=jnp.float32) + b2_ref[...]
    o_ref[...] = x1 + modt[:, 3 * D:4 * D] * mlp


def _tc2(x, t0, t1, modt, Wp, bp, E8, W1, b1, W2, b2):
    BN = 1000
    grid = (N // BN,)
    return pl.pallas_call(
        _tc2_body,
        grid=grid,
        in_specs=[
            pl.BlockSpec((BN, D), lambda i: (i, 0)),
            pl.BlockSpec((BN, AW), lambda i: (i, 0)),
            pl.BlockSpec((BN, AW), lambda i: (i, 0)),
            pl.BlockSpec((BN, 4 * D), lambda i: (i, 0)),
            pl.BlockSpec((D, D), lambda i: (0, 0)),
            pl.BlockSpec((1, D), lambda i: (0, 0)),
            pl.BlockSpec((H, D), lambda i: (0, 0)),
            pl.BlockSpec((D, FF), lambda i: (0, 0)),
            pl.BlockSpec((1, FF), lambda i: (0, 0)),
            pl.BlockSpec((FF, D), lambda i: (0, 0)),
            pl.BlockSpec((1, D), lambda i: (0, 0)),
        ],
        out_specs=pl.BlockSpec((BN, D), lambda i: (i, 0)),
        out_shape=jax.ShapeDtypeStruct((N, D), jnp.float32),
    )(x, t0, t1, modt, Wp, bp, E8, W1, b1, W2, b2)


def kernel(x, edge_index, edge_attr, c, Wq, bq, Wkv, bkv, Wp, bp,
           W1, b1, W2, b2, Wa, ba):
    f32 = jnp.float32
    Wkvh = Wkv[:D]
    Wkve = Wkv[D:]
    hs = jnp.arange(H)
    Wq2e = jnp.zeros((D, H * ED), f32)
    Wvecat = jnp.zeros((ED, D), f32)
    E8 = jnp.zeros((H, D), f32)
    for h in range(H):
        Ke = lax.dynamic_slice(Wkve, (0, h * 2 * DH), (ED, DH))
        Ve = lax.dynamic_slice(Wkve, (0, h * 2 * DH + DH), (ED, DH))
        Wq2e = lax.dynamic_update_slice(Wq2e, Ke.T, (h * DH, h * ED))
        Wvecat = lax.dynamic_update_slice(Wvecat, Ve, (0, h * DH))
        E8 = lax.dynamic_update_slice(E8, jnp.ones((1, DH), f32), (h, h * DH))

    ba2 = ba.reshape(1, 6 * D)
    bq2 = bq.reshape(1, D)
    bkv2 = bkv.reshape(1, 2 * D)
    bp2 = bp.reshape(1, D)
    b12 = b1.reshape(1, FF)
    b22 = b2.reshape(1, D)

    q, qe, nkv, modt = _tc1(x, c, Wa, ba2, Wq, bq2, Wkvh, bkv2, Wq2e)

    src = edge_index[0]
    dst = edge_index[1]
    e0, e1 = _tcev(edge_attr, Wvecat)
    zeros = jnp.zeros((N, AW), f32)
    bf16 = jnp.bfloat16

    def _f32bits(a):
        return lax.bitcast_convert_type(a, bf16).reshape(a.shape[0], -1)

    qc0 = jnp.concatenate([q[:, :128].astype(bf16), _f32bits(qe[:, :64])], axis=1)
    qc1 = jnp.concatenate([q[:, 128:].astype(bf16), _f32bits(qe[:, 64:])], axis=1)
    sd = jnp.pad(edge_index.reshape(2, NS, NWIN, W),
                 ((0, 0), (0, 0), (0, 0), (0, PW - W)))
    sd = sd.transpose(1, 2, 0, 3).reshape(NS * NWIN, 2, PW)
    t0, t1 = _sc_edge(sd, edge_attr, e0, e1,
                      qc0, qc1,
                      nkv[:, :256].astype(bf16), nkv[:, 256:].astype(bf16),
                      zeros)

    perm = jnp.arange(H * DH).reshape(H, DH // 2, 2).transpose(0, 2, 1).reshape(-1)
    Wp_perm = Wp[perm]
    return _tc2(x, t0, t1, modt, Wp_perm, bp2, E8, W1, b12, W2, b22)
